# baseline (speedup 1.0000x reference)
import functools

import jax
import jax.numpy as jnp
from jax import lax
from jax.experimental import pallas as pl
from jax.experimental.pallas import tpu as pltpu
from jax.experimental.pallas import tpu_sc as plsc


def _make_sc_kernel(V, D, B):
    info = plsc.get_sparse_core_info()
    NC, NS, L = info.num_cores, info.num_subcores, info.num_lanes
    NW = NC * NS
    assert B % (8 * NW) == 0 and D == 64 and L == 16
    b_per_w = B // NW
    n_groups = b_per_w // L
    mesh = plsc.VectorSubcoreMesh(core_axis_name="c", subcore_axis_name="s")

    @functools.partial(
        pl.kernel,
        mesh=mesh,
        compiler_params=pltpu.CompilerParams(needs_layout_passes=False),
        out_type=jax.ShapeDtypeStruct((2, B), jnp.float32),
        scratch_types=[
            pltpu.VMEM((b_per_w,), jnp.int32),
            pltpu.VMEM((b_per_w, D), jnp.float32),
            pltpu.VMEM((256,), jnp.float32),
            pltpu.VMEM((2, b_per_w), jnp.float32),
            pltpu.SemaphoreType.DMA,
        ],
    )
    def sc_k(idx_hbm, table_hbm, wb_hbm, out_hbm, idx_s, rows_v, wb_v, out_v,
             sem):
        wid = lax.axis_index("s") * NC + lax.axis_index("c")
        base = wid * b_per_w
        pltpu.sync_copy(idx_hbm.at[pl.ds(base, b_per_w)], idx_s)
        pltpu.sync_copy(wb_hbm, wb_v)

        def fire_batch(g, c):
            iv = idx_s[pl.ds(g * L, L)]
            for k in range(L):
                pltpu.async_copy(table_hbm.at[iv[k]], rows_v.at[g * L + k],
                                 sem)
            return c

        lax.fori_loop(0, n_groups, fire_batch, 0)

        def drain(j, c):
            pltpu.make_async_copy(table_hbm.at[0], rows_v.at[0], sem).wait()
            return c

        lax.fori_loop(0, b_per_w, drain, 0)

        bvec = wb_v[pl.ds(128, 16)]
        b0 = bvec[0]
        b1 = bvec[1]

        def compute(g, c):
            rowbase = g * L + lax.iota(jnp.int32, L)
            acc0 = jnp.zeros((L,), jnp.float32)
            acc1 = jnp.zeros((L,), jnp.float32)
            for hc in range(D // L):
                w0v = wb_v[pl.ds(16 * hc, 16)]
                w1v = wb_v[pl.ds(64 + 16 * hc, 16)]
                for k in range(L):
                    h = 16 * hc + k
                    col = jnp.full((L,), h, jnp.int32)
                    v = plsc.load_gather(rows_v, [rowbase, col])
                    e = jnp.exp(v + v)
                    t = 1.0 - 2.0 / (e + 1.0)
                    acc0 = acc0 + t * w0v[k]
                    acc1 = acc1 + t * w1v[k]
            out_v[0, pl.ds(g * L, L)] = acc0 + b0
            out_v[1, pl.ds(g * L, L)] = acc1 + b1
            return c

        lax.fori_loop(0, n_groups, compute, 0)
        pltpu.sync_copy(out_v, out_hbm.at[:, pl.ds(base, b_per_w)])

    return sc_k


def kernel(state_index, emb_table, lin_w, lin_b):
    V, D = emb_table.shape
    B = state_index.shape[0]
    idx = state_index.astype(jnp.int32)
    wb = jnp.concatenate(
        [lin_w.reshape(-1), lin_b, jnp.zeros((126,), jnp.float32)])
    out = _make_sc_kernel(V, D, B)(idx, emb_table, wb)
    return out.T


# speedup vs baseline: 1.0033x; 1.0033x over previous
import functools

import jax
import jax.numpy as jnp
from jax import lax
from jax.experimental import pallas as pl
from jax.experimental.pallas import tpu as pltpu
from jax.experimental.pallas import tpu_sc as plsc


def _make_sc_kernel(V, D, B):
    info = plsc.get_sparse_core_info()
    NC, NS, L = info.num_cores, info.num_subcores, info.num_lanes
    NW = NC * NS
    assert B % (8 * NW) == 0 and D == 64 and L == 16
    b_per_w = B // NW
    n_groups = b_per_w // L
    mesh = plsc.VectorSubcoreMesh(core_axis_name="c", subcore_axis_name="s")

    @functools.partial(
        pl.kernel,
        mesh=mesh,
        compiler_params=pltpu.CompilerParams(needs_layout_passes=False),
        out_type=jax.ShapeDtypeStruct((2, B), jnp.float32),
        scratch_types=[
            pltpu.VMEM((b_per_w,), jnp.int32),
            pltpu.VMEM((b_per_w, D), jnp.float32),
            pltpu.VMEM((256,), jnp.float32),
            pltpu.VMEM((2, b_per_w), jnp.float32),
            pltpu.SemaphoreType.DMA,
        ],
    )
    def sc_k(idx_hbm, table_hbm, wb_hbm, out_hbm, idx_s, rows_v, wb_v, out_v,
             sem):
        wid = lax.axis_index("s") * NC + lax.axis_index("c")
        base = wid * b_per_w
        pltpu.sync_copy(idx_hbm.at[pl.ds(base, b_per_w)], idx_s)
        pltpu.sync_copy(wb_hbm, wb_v)

        def fire_batch(g, c):
            iv = idx_s[pl.ds(g * L, L)]
            for k in range(L):
                pltpu.async_copy(table_hbm.at[iv[k]], rows_v.at[g * L + k],
                                 sem)
            return c

        lax.fori_loop(0, n_groups, fire_batch, 0)
        pltpu.make_async_copy(table_hbm.at[pl.ds(0, b_per_w)], rows_v,
                              sem).wait()

        bvec = wb_v[pl.ds(128, 16)]
        b0 = bvec[0]
        b1 = bvec[1]
        lanes = lax.iota(jnp.int32, L)

        def compute(g, c):
            rowbase = g * L + lanes
            acc0 = jnp.zeros((L,), jnp.float32)
            acc1 = jnp.zeros((L,), jnp.float32)
            for hc in range(D // L):
                w0v = wb_v[pl.ds(16 * hc, 16)]
                w1v = wb_v[pl.ds(64 + 16 * hc, 16)]
                for k in range(L):
                    h = 16 * hc + k
                    col = jnp.full((L,), h, jnp.int32)
                    v = plsc.load_gather(rows_v, [rowbase, col])
                    e = jnp.exp(v + v)
                    t = 1.0 - 2.0 / (e + 1.0)
                    acc0 = acc0 + t * w0v[k]
                    acc1 = acc1 + t * w1v[k]
            out_v[0, pl.ds(g * L, L)] = acc0 + b0
            out_v[1, pl.ds(g * L, L)] = acc1 + b1
            return c

        lax.fori_loop(0, n_groups, compute, 0)
        pltpu.sync_copy(out_v, out_hbm.at[:, pl.ds(base, b_per_w)])

    return sc_k


def kernel(state_index, emb_table, lin_w, lin_b):
    V, D = emb_table.shape
    B = state_index.shape[0]
    idx = state_index.astype(jnp.int32)
    wb = jnp.concatenate(
        [lin_w.reshape(-1), lin_b, jnp.zeros((126,), jnp.float32)])
    out = _make_sc_kernel(V, D, B)(idx, emb_table, wb)
    return out.T


# speedup vs baseline: 3.3707x; 3.3597x over previous
import functools

import jax
import jax.numpy as jnp
from jax import lax
from jax.experimental import pallas as pl
from jax.experimental.pallas import tpu as pltpu
from jax.experimental.pallas import tpu_sc as plsc


def _tc_body(tt_ref, w_ref, b_ref, out0_ref, out1_ref):
    t = jnp.tanh(tt_ref[...])
    acc = lax.dot_general(w_ref[...], t, (((1,), (0,)), ((), ())),
                          preferred_element_type=jnp.float32)
    out0_ref[...] = acc[0, :] + b_ref[0]
    out1_ref[...] = acc[1, :] + b_ref[1]


def _make_sc_gather(V, B):
    info = plsc.get_sparse_core_info()
    NC, NS = info.num_cores, info.num_subcores
    NW = NC * NS
    assert B % (8 * NW) == 0
    b_per_w = B // NW
    mesh = plsc.VectorSubcoreMesh(core_axis_name="c", subcore_axis_name="s")

    @functools.partial(
        pl.kernel,
        mesh=mesh,
        compiler_params=pltpu.CompilerParams(use_tc_tiling_on_sc=False),
        out_type=jax.ShapeDtypeStruct((2, B), jnp.float32),
        scratch_types=[
            pltpu.VMEM((b_per_w,), jnp.int32),
            pltpu.VMEM((2, b_per_w), jnp.float32),
            pltpu.SemaphoreType.DMA,
        ],
    )
    def sc_k(idx_hbm, l0_hbm, l1_hbm, out_hbm, idx_v, g_v, sem):
        wid = lax.axis_index("s") * NC + lax.axis_index("c")
        base = wid * b_per_w
        pltpu.sync_copy(idx_hbm.at[pl.ds(base, b_per_w)], idx_v)
        pltpu.async_copy(l0_hbm.at[idx_v], g_v.at[0], sem).wait()
        pltpu.async_copy(l1_hbm.at[idx_v], g_v.at[1], sem).wait()
        pltpu.sync_copy(g_v, out_hbm.at[:, pl.ds(base, b_per_w)])

    return sc_k


def kernel(state_index, emb_table, lin_w, lin_b):
    V, D = emb_table.shape
    B = state_index.shape[0]
    idx = state_index.astype(jnp.int32)
    table_t = emb_table.T

    CB = 16384
    grid = pl.cdiv(V, CB)
    l0, l1 = pl.pallas_call(
        _tc_body,
        grid=(grid,),
        in_specs=[
            pl.BlockSpec((D, CB), lambda i: (0, i)),
            pl.BlockSpec((2, D), lambda i: (0, 0)),
            pl.BlockSpec(memory_space=pltpu.SMEM),
        ],
        out_specs=[
            pl.BlockSpec((CB,), lambda i: (i,)),
            pl.BlockSpec((CB,), lambda i: (i,)),
        ],
        out_shape=[
            jax.ShapeDtypeStruct((V,), jnp.float32),
            jax.ShapeDtypeStruct((V,), jnp.float32),
        ],
    )(table_t, lin_w, lin_b)

    out = _make_sc_gather(V, B)(idx, l0, l1)
    return out.T


# speedup vs baseline: 3.9011x; 1.1574x over previous
import functools

import jax
import jax.numpy as jnp
from jax import lax
from jax.experimental import pallas as pl
from jax.experimental.pallas import tpu as pltpu
from jax.experimental.pallas import tpu_sc as plsc


def _tc_body(tt_ref, w_ref, b_ref, out0_ref, out1_ref):
    t = jnp.tanh(tt_ref[...])
    acc = lax.dot_general(w_ref[...], t, (((1,), (0,)), ((), ())),
                          preferred_element_type=jnp.float32)
    out0_ref[...] = acc[0, :] + b_ref[0]
    out1_ref[...] = acc[1, :] + b_ref[1]


def _make_sc_gather(V, B):
    info = plsc.get_sparse_core_info()
    NC, NS = info.num_cores, info.num_subcores
    NW = NC * NS
    assert B % (8 * NW) == 0
    b_per_w = B // NW
    mesh = plsc.VectorSubcoreMesh(core_axis_name="c", subcore_axis_name="s")

    @functools.partial(
        pl.kernel,
        mesh=mesh,
        compiler_params=pltpu.CompilerParams(use_tc_tiling_on_sc=False),
        out_type=jax.ShapeDtypeStruct((2, B), jnp.float32),
        scratch_types=[
            pltpu.VMEM((b_per_w,), jnp.int32),
            pltpu.VMEM((2, b_per_w), jnp.float32),
            pltpu.SemaphoreType.DMA,
        ],
    )
    def sc_k(idx_hbm, l0_hbm, l1_hbm, out_hbm, idx_v, g_v, sem):
        wid = lax.axis_index("s") * NC + lax.axis_index("c")
        base = wid * b_per_w
        pltpu.sync_copy(idx_hbm.at[pl.ds(base, b_per_w)], idx_v)
        pltpu.async_copy(l0_hbm.at[idx_v], g_v.at[0], sem).wait()
        pltpu.async_copy(l1_hbm.at[idx_v], g_v.at[1], sem).wait()
        pltpu.sync_copy(g_v, out_hbm.at[:, pl.ds(base, b_per_w)])

    return sc_k


def kernel(state_index, emb_table, lin_w, lin_b):
    V, D = emb_table.shape
    B = state_index.shape[0]
    idx = state_index.astype(jnp.int32)
    table_t = emb_table.T

    CB = 32768
    grid = pl.cdiv(V, CB)
    l0, l1 = pl.pallas_call(
        _tc_body,
        grid=(grid,),
        in_specs=[
            pl.BlockSpec((D, CB), lambda i: (0, i)),
            pl.BlockSpec((2, D), lambda i: (0, 0)),
            pl.BlockSpec(memory_space=pltpu.SMEM),
        ],
        out_specs=[
            pl.BlockSpec((CB,), lambda i: (i,)),
            pl.BlockSpec((CB,), lambda i: (i,)),
        ],
        out_shape=[
            jax.ShapeDtypeStruct((V,), jnp.float32),
            jax.ShapeDtypeStruct((V,), jnp.float32),
        ],
    )(table_t, lin_w, lin_b)

    out = _make_sc_gather(V, B)(idx, l0, l1)
    return out.T
